# Initial kernel scaffold; baseline (speedup 1.0000x reference)
import jax
import jax.numpy as jnp
from jax import lax
from jax.experimental import pallas as pl
from jax.experimental.pallas import tpu as pltpu
from jax.experimental.pallas import tpu_sc as plsc

_F = 26
_V = 100000
_E = 32
_B = 16384
_TOT = _B * _F
_G = 128


def _make_body(nc, ns):
    nw = nc * ns
    pw = _TOT // nw
    nch = pw // _G

    def body(tab_hbm, xe_hbm, off_hbm, out_hbm, xe_v, off_v, idx_v, buf_v, sem):
        wid = lax.axis_index("s") * nc + lax.axis_index("c")
        row0 = wid * nch
        pltpu.sync_copy(xe_hbm.at[pl.ds(row0, nch)], xe_v)
        pltpu.sync_copy(off_hbm, off_v)

        def compute(j, carry):
            for i in range(_G // 16):
                s = pl.ds(i * 16, 16)
                idx_v[j, s] = xe_v[j, s] + off_v[j, s]
            return carry

        lax.fori_loop(0, nch, compute, 0)

        def gather(j, carry):
            pltpu.async_copy(tab_hbm.at[idx_v.at[j]], buf_v, sem).wait()
            pltpu.sync_copy(buf_v, out_hbm.at[pl.ds((row0 + j) * _G, _G)])
            return carry

        lax.fori_loop(0, nch, gather, 0)

    return body, nch


def kernel(xe, tables):
    info = plsc.get_sparse_core_info()
    nc, ns = info.num_cores, info.num_subcores
    body, nch = _make_body(nc, ns)

    tab = tables.reshape(_F * _V, _E)
    xe_flat = xe.reshape(_TOT // _G, _G)
    off = jnp.tile(jnp.arange(_F, dtype=jnp.int32) * _V,
                   (_TOT // (nc * ns)) // _F).reshape(nch, _G)

    mesh = plsc.VectorSubcoreMesh(core_axis_name="c", subcore_axis_name="s")
    k = pl.kernel(
        body,
        out_type=jax.ShapeDtypeStruct((_TOT, _E), jnp.float32),
        mesh=mesh,
        scratch_types=[
            pltpu.VMEM((nch, _G), jnp.int32),
            pltpu.VMEM((nch, _G), jnp.int32),
            pltpu.VMEM((nch, _G), jnp.int32),
            pltpu.VMEM((_G, _E), jnp.float32),
            pltpu.SemaphoreType.DMA,
        ],
    )
    out = k(tab, xe_flat, off)
    return out.reshape(_B, _F * _E)


# speedup vs baseline: 1.1468x; 1.1468x over previous
import jax
import jax.numpy as jnp
from jax import lax
from jax.experimental import pallas as pl
from jax.experimental.pallas import tpu as pltpu
from jax.experimental.pallas import tpu_sc as plsc

_F = 26
_V = 100000
_E = 32
_B = 16384
_TOT = _B * _F
_G = 128


def _make_body(nc, ns):
    nw = nc * ns
    pw = _TOT // nw
    nch = pw // _G

    def body(tab_hbm, xe_hbm, off_hbm, out_hbm, xe_v, off_v, idx_v, buf_v, sem):
        wid = lax.axis_index("s") * nc + lax.axis_index("c")
        row0 = wid * nch
        pltpu.sync_copy(xe_hbm.at[pl.ds(row0, nch)], xe_v)
        pltpu.sync_copy(off_hbm, off_v)

        def compute(j, carry):
            for i in range(_G // 16):
                s = pl.ds(i * 16, 16)
                idx_v[j, s] = xe_v[j, s] + off_v[j, s]
            return carry

        lax.fori_loop(0, nch, compute, 0)

        def gather(j, carry):
            pltpu.async_copy(tab_hbm.at[idx_v.at[j]], buf_v, sem).wait()
            pltpu.sync_copy(buf_v, out_hbm.at[pl.ds((row0 + j) * _G, _G)])
            return carry

        lax.fori_loop(0, nch, gather, 0)

    return body, nch


def kernel(xe, tables):
    info = plsc.get_sparse_core_info()
    nc, ns = info.num_cores, info.num_subcores
    body, nch = _make_body(nc, ns)

    tab = tables.reshape(_F * _V, _E)
    xe_flat = xe.reshape(_TOT // _G, _G)
    off = jnp.tile(jnp.arange(_F, dtype=jnp.int32) * _V,
                   (_TOT // (nc * ns)) // _F).reshape(nch, _G)

    mesh = plsc.VectorSubcoreMesh(core_axis_name="c", subcore_axis_name="s")
    k = pl.kernel(
        body,
        out_type=jax.ShapeDtypeStruct((_TOT, _E), jnp.float32),
        mesh=mesh,
        scratch_types=[
            pltpu.VMEM((nch, _G), jnp.int32),
            pltpu.VMEM((nch, _G), jnp.int32),
            pltpu.VMEM((nch, _G), jnp.int32),
            pltpu.VMEM((_G, _E), jnp.float32),
            pltpu.SemaphoreType.DMA,
        ],
        compiler_params=pltpu.CompilerParams(use_tc_tiling_on_sc=False),
    )
    out = k(tab, xe_flat, off)
    return out.reshape(_B, _F * _E)


# speedup vs baseline: 1.2133x; 1.0579x over previous
import jax
import jax.numpy as jnp
from jax import lax
from jax.experimental import pallas as pl
from jax.experimental.pallas import tpu as pltpu
from jax.experimental.pallas import tpu_sc as plsc

_F = 26
_V = 100000
_E = 32
_B = 16384
_TOT = _B * _F
_G = 128


def _make_body(nc, ns):
    nw = nc * ns
    pw = _TOT // nw
    nch = pw // _G

    nb = 8
    nrounds = nch // nb

    def body(tab_hbm, xe_hbm, off_hbm, out_hbm, xe_v, off_v, idx_v, buf_v,
             gsem, ssem):
        wid = lax.axis_index("s") * nc + lax.axis_index("c")
        row0 = wid * nch
        pltpu.sync_copy(xe_hbm.at[pl.ds(row0, nch)], xe_v)
        pltpu.sync_copy(off_hbm, off_v)

        def compute(j, carry):
            for i in range(_G // 16):
                s = pl.ds(i * 16, 16)
                idx_v[j, s] = xe_v[j, s] + off_v[j, s]
            return carry

        lax.fori_loop(0, nch, compute, 0)

        def gather_start(j, b):
            pltpu.async_copy(tab_hbm.at[idx_v.at[j]], buf_v.at[b], gsem.at[b])

        def gather_wait(b):
            pltpu.make_async_copy(tab_hbm.at[idx_v.at[0]], buf_v.at[b],
                                  gsem.at[b]).wait()

        def store_start(j, b):
            pltpu.async_copy(buf_v.at[b], out_hbm.at[pl.ds((row0 + j) * _G, _G)],
                             ssem.at[b])

        def store_wait(b):
            pltpu.make_async_copy(buf_v.at[b], out_hbm.at[pl.ds(0, _G)],
                                  ssem.at[b]).wait()

        for b in range(nb):
            gather_start(b, b)

        def round_fn(r, carry):
            j0 = r * nb
            for b in range(nb):
                gather_wait(b)
                store_start(j0 + b, b)
            for b in range(nb):
                @pl.when(r < nrounds - 1)
                def _():
                    store_wait(b)
                    gather_start(j0 + nb + b, b)
            return carry

        lax.fori_loop(0, nrounds, round_fn, 0)

        for b in range(nb):
            store_wait(b)

    return body, nch, nb


def kernel(xe, tables):
    info = plsc.get_sparse_core_info()
    nc, ns = info.num_cores, info.num_subcores
    body, nch, nb = _make_body(nc, ns)

    tab = tables.reshape(_F * _V, _E)
    xe_flat = xe.reshape(_TOT // _G, _G)
    off = jnp.tile(jnp.arange(_F, dtype=jnp.int32) * _V,
                   (_TOT // (nc * ns)) // _F).reshape(nch, _G)

    mesh = plsc.VectorSubcoreMesh(core_axis_name="c", subcore_axis_name="s")
    k = pl.kernel(
        body,
        out_type=jax.ShapeDtypeStruct((_TOT, _E), jnp.float32),
        mesh=mesh,
        scratch_types=[
            pltpu.VMEM((nch, _G), jnp.int32),
            pltpu.VMEM((nch, _G), jnp.int32),
            pltpu.VMEM((nch, _G), jnp.int32),
            pltpu.VMEM((nb, _G, _E), jnp.float32),
            pltpu.SemaphoreType.DMA((nb,)),
            pltpu.SemaphoreType.DMA((nb,)),
        ],
        compiler_params=pltpu.CompilerParams(use_tc_tiling_on_sc=False),
    )
    out = k(tab, xe_flat, off)
    return out.reshape(_B, _F * _E)
